# baseline (speedup 1.0000x reference)
import functools

import jax
import jax.numpy as jnp
from jax import lax
from jax.experimental import pallas as pl
from jax.experimental.pallas import tpu as pltpu
from jax.experimental.pallas import tpu_sc as plsc

N_NODES = 50000
NP = 2 * N_NODES
NP2 = 100352
SLICE = NP2 // 16
NW = 32
S = 512
GROUPS = S // 16
R1 = S // 128
PROWS = 8 * R1
CG_ITERS_K = 32


def _make_sc_matvec(nsub):
    mesh = plsc.VectorSubcoreMesh(core_axis_name="c", subcore_axis_name="s")
    half = nsub // 2

    @functools.partial(
        pl.kernel,
        out_type=jax.ShapeDtypeStruct((2, NP2), jnp.float32),
        mesh=mesh,
        compiler_params=pltpu.CompilerParams(needs_layout_passes=False),
        scratch_types=[
            pltpu.VMEM((NP,), jnp.float32),
            pltpu.VMEM((PROWS, 128), jnp.int32),
            pltpu.VMEM((S,), jnp.float32),
            pltpu.VMEM((S,), jnp.float32),
            pltpu.VMEM((PROWS, 128), jnp.int32),
            pltpu.VMEM((S,), jnp.float32),
            pltpu.VMEM((S,), jnp.float32),
            pltpu.VMEM((SLICE,), jnp.float32),
            pltpu.VMEM_SHARED((NP2,), jnp.float32),
            pltpu.SemaphoreType.DMA,
            pltpu.SemaphoreType.DMA,
        ],
    )
    def sc_matvec(v_hbm, pk_hbm, out_hbm,
                  v_v, pk_a, v0_a, v1_a, pk_b, v0_b, v1_b,
                  buf_v, acc, ssem, csem):
        cid = lax.axis_index("c")
        sid = lax.axis_index("s")
        wid = cid * 16 + sid

        def fire_stage(sub, pk_v):
            row = pl.multiple_of((wid * nsub + sub) * PROWS, 8)
            pltpu.async_copy(pk_hbm.at[pl.ds(row, PROWS)], pk_v, ssem)

        def drain_stage():
            pltpu.make_async_copy(pk_hbm.at[pl.ds(0, PROWS)], pk_a,
                                  ssem).wait()

        def compute(pk_v, v0_v, v1_v):
            def g_body(g, cc):
                o = g * 16
                r = g // 8
                c = (g % 8) * 16
                jv = pk_v[r, pl.ds(c, 16)]
                vj0 = plsc.load_gather(v_v, [jv])
                vj1 = plsc.load_gather(v_v, [jv + 1])
                h00 = plsc.bitcast(pk_v[3 * R1 + r, pl.ds(c, 16)],
                                   jnp.float32)
                h01 = plsc.bitcast(pk_v[4 * R1 + r, pl.ds(c, 16)],
                                   jnp.float32)
                h10 = plsc.bitcast(pk_v[5 * R1 + r, pl.ds(c, 16)],
                                   jnp.float32)
                h11 = plsc.bitcast(pk_v[6 * R1 + r, pl.ds(c, 16)],
                                   jnp.float32)
                v0_v[pl.ds(o, 16)] = h00 * vj0 + h01 * vj1
                v1_v[pl.ds(o, 16)] = h10 * vj0 + h11 * vj1
                return cc
            lax.fori_loop(0, GROUPS, g_body, 0)

        def fire_scatter(pk_v, v0_v, v1_v):
            def sb(k, cc):
                pltpu.async_copy(v0_v.at[pl.ds(k * 128, 128)],
                                 acc.at[pk_v.at[R1 + k]], csem, add=True)
                pltpu.async_copy(v1_v.at[pl.ds(k * 128, 128)],
                                 acc.at[pk_v.at[2 * R1 + k]], csem, add=True)
                return cc
            lax.fori_loop(0, R1, sb, 0)

        def drain_scatter():
            cp = pltpu.make_async_copy(v_hbm.at[pl.ds(0, S)], v0_a, csem)
            cp.wait()
            cp.wait()

        def zbody(k, c):
            buf_v[pl.ds(k * 16, 16)] = jnp.zeros((16,), jnp.float32)
            return c
        lax.fori_loop(0, SLICE // 16, zbody, 0)
        pltpu.sync_copy(buf_v, acc.at[pl.ds(sid * SLICE, SLICE)])
        pltpu.sync_copy(v_hbm.at[pl.ds(0, NP)], v_v)
        plsc.subcore_barrier()

        fire_stage(0, pk_a)

        def body(p, c):
            sub0 = p * 2
            drain_stage()
            compute(pk_a, v0_a, v1_a)

            @pl.when(p > 0)
            def _():
                drain_scatter()
            fire_stage(sub0 + 1, pk_b)
            fire_scatter(pk_a, v0_a, v1_a)
            drain_stage()
            compute(pk_b, v0_b, v1_b)
            drain_scatter()

            @pl.when(p + 1 < half)
            def _():
                fire_stage(sub0 + 2, pk_a)
            fire_scatter(pk_b, v0_b, v1_b)
            return c
        lax.fori_loop(0, half, body, 0)
        drain_scatter()

        plsc.subcore_barrier()
        pltpu.sync_copy(acc.at[pl.ds(sid * SLICE, SLICE)], buf_v)
        pltpu.sync_copy(buf_v, out_hbm.at[cid, pl.ds(sid * SLICE, SLICE)])

    return sc_matvec


def _tc_init_body(apk2_ref, b_ref, r_ref, p_ref):
    r = apk2_ref[0] + apk2_ref[1] - b_ref[...]
    r_ref[...] = r
    p_ref[...] = -r


def _tc_init(apk2, b):
    shp = jax.ShapeDtypeStruct((NP2 // 128, 128), jnp.float32)
    return pl.pallas_call(_tc_init_body, out_shape=(shp, shp))(apk2, b)


def _tc_update_body(apk2_ref, pk_ref, rk_ref, xk_ref, xo_ref, ro_ref, po_ref):
    apk = apk2_ref[0] + apk2_ref[1]
    rk = rk_ref[...]
    pk = pk_ref[...]
    rkrk = jnp.sum(rk * rk)
    alpha = rkrk / jnp.sum(pk * apk)
    xo_ref[...] = xk_ref[...] + alpha * pk
    rn = rk + alpha * apk
    ro_ref[...] = rn
    beta = jnp.sum(rn * rn) / rkrk
    po_ref[...] = -rn + beta * pk


def _tc_update(apk2, pk, rk, xk):
    shp = jax.ShapeDtypeStruct((NP2 // 128, 128), jnp.float32)
    return pl.pallas_call(_tc_update_body,
                          out_shape=(shp, shp, shp))(apk2, pk, rk, xk)


def kernel(H, B, x0, i, j):
    f32 = jnp.float32
    i32 = jnp.int32
    e = i.shape[0]
    per_w = -(-e // (NW * 2 * S)) * 2 * S
    e_pad = per_w * NW
    nsub = per_w // S
    pad = e_pad - e
    nchunk = e_pad // S

    def padded(x, fill):
        return jnp.concatenate([x, jnp.full((pad,), fill, x.dtype)])

    cast = lambda h: lax.bitcast_convert_type(h, i32)
    i2 = padded(2 * i, NP2 - 2)
    packed = jnp.stack([
        padded(2 * j, 0).reshape(nchunk, S),
        i2.reshape(nchunk, S),
        (i2 + 1).reshape(nchunk, S),
        padded(cast(H[:, 0, 0]), 0).reshape(nchunk, S),
        padded(cast(H[:, 0, 1]), 0).reshape(nchunk, S),
        padded(cast(H[:, 1, 0]), 0).reshape(nchunk, S),
        padded(cast(H[:, 1, 1]), 0).reshape(nchunk, S),
        jnp.zeros((nchunk, S), i32),
    ], axis=1).reshape(nchunk * PROWS, 128)

    bp = jnp.pad(B, (0, NP2 - NP)).reshape(NP2 // 128, 128)
    x0p = jnp.pad(x0, (0, NP2 - NP))

    sc_matvec = _make_sc_matvec(nsub)

    def matvec(v):
        out = sc_matvec(v, packed)
        return out.reshape(2, NP2 // 128, 128)

    rk, pk = _tc_init(matvec(x0p), bp)
    xk = x0p.reshape(NP2 // 128, 128)

    def body(carry, _):
        xk, rk, pk = carry
        apk2 = matvec(pk.reshape(NP2))
        xk, rk, pk = _tc_update(apk2, pk, rk, xk)
        return (xk, rk, pk), None

    (xk, rk, pk), _ = lax.scan(body, (xk, rk, pk), None, length=CG_ITERS_K)
    return xk.reshape(NP2)[:NP]
